# baseline (speedup 1.0000x reference)
import functools
import jax
import jax.numpy as jnp
from jax import lax
from jax.experimental import pallas as pl
from jax.experimental.pallas import tpu as pltpu
from jax.experimental.pallas import tpu_sc as plsc

_N_PAPER = 100000
_N_AUTHOR = 50000
_N_FIELD = 10000
_EMBED = 128
_TOTAL = _N_PAPER + _N_AUTHOR + _N_FIELD
_NW = 32
_ROWS_PER_W = _TOTAL // _NW
_CH = 200
_NCH = _ROWS_PER_W // _CH


def _copy_range(src, out, sbase, obase, bufs, sin, son):
    pending = [None, None]
    for c in range(_NCH):
        b = c % 2
        if pending[b] is not None:
            pending[b].wait()
        incp = pltpu.make_async_copy(
            src.at[pl.ds(sbase + c * _CH, _CH)], bufs[b], sin[b])
        incp.start()
        incp.wait()
        ocp = pltpu.make_async_copy(
            bufs[b], out.at[pl.ds(obase + c * _CH, _CH)], son[b])
        ocp.start()
        pending[b] = ocp
    for b in (0, 1):
        if pending[b] is not None:
            pending[b].wait()


def _sc_body(p_hbm, a_hbm, f_hbm, out_hbm, buf0, buf1, si0, si1, so0, so1):
    wid = lax.axis_index("s") * 2 + lax.axis_index("c")
    bufs = (buf0, buf1)
    sin = (si0, si1)
    son = (so0, so1)
    pb_w = _N_PAPER // _ROWS_PER_W
    ab_w = _N_AUTHOR // _ROWS_PER_W

    @pl.when(wid < pb_w)
    def _():
        _copy_range(p_hbm, out_hbm, wid * _ROWS_PER_W, wid * _ROWS_PER_W,
                    bufs, sin, son)

    @pl.when(jnp.logical_and(wid >= pb_w, wid < pb_w + ab_w))
    def _():
        r = (wid - pb_w) * _ROWS_PER_W
        _copy_range(a_hbm, out_hbm, r, _N_PAPER + r, bufs, sin, son)

    @pl.when(wid >= pb_w + ab_w)
    def _():
        r = (wid - pb_w - ab_w) * _ROWS_PER_W
        _copy_range(f_hbm, out_hbm, r, _N_PAPER + _N_AUTHOR + r,
                    bufs, sin, son)


def kernel(embed_paper, embed_author, embed_field):
    k = pl.kernel(
        _sc_body,
        out_type=jax.ShapeDtypeStruct((_TOTAL, _EMBED), jnp.float32),
        mesh=plsc.VectorSubcoreMesh(core_axis_name="c", subcore_axis_name="s"),
        scratch_types=[
            pltpu.VMEM((_CH, _EMBED), jnp.float32),
            pltpu.VMEM((_CH, _EMBED), jnp.float32),
            pltpu.SemaphoreType.DMA,
            pltpu.SemaphoreType.DMA,
            pltpu.SemaphoreType.DMA,
            pltpu.SemaphoreType.DMA,
        ],
    )
    return k(embed_paper, embed_author, embed_field)


# speedup vs baseline: 1.0647x; 1.0647x over previous
import jax
import jax.numpy as jnp
from jax import lax
from jax.experimental import pallas as pl
from jax.experimental.pallas import tpu as pltpu
from jax.experimental.pallas import tpu_sc as plsc

_N_PAPER = 100000
_N_AUTHOR = 50000
_N_FIELD = 10000
_EMBED = 128
_TOTAL = _N_PAPER + _N_AUTHOR + _N_FIELD
_NW = 32
_ROWS_PER_W = _TOTAL // _NW
_CH = 200
_NCH = _ROWS_PER_W // _CH
_NBUF = 4
_LOOKAHEAD = 2


def _copy_range(src, out, sbase, obase, bufs, sin, son):
    in_cp = [None] * _NBUF
    out_cp = [None] * _NBUF

    def make_in(c):
        b = c % _NBUF
        cp = pltpu.make_async_copy(
            src.at[pl.ds(sbase + c * _CH, _CH)], bufs[b], sin[b])
        cp.start()
        in_cp[b] = cp

    for c in range(min(_LOOKAHEAD, _NCH)):
        make_in(c)
    for c in range(_NCH):
        b = c % _NBUF
        in_cp[b].wait()
        ocp = pltpu.make_async_copy(
            bufs[b], out.at[pl.ds(obase + c * _CH, _CH)], son[b])
        ocp.start()
        out_cp[b] = ocp
        n = c + _LOOKAHEAD
        if n < _NCH:
            nb = n % _NBUF
            if out_cp[nb] is not None:
                out_cp[nb].wait()
                out_cp[nb] = None
            make_in(n)
    for b in range(_NBUF):
        if out_cp[b] is not None:
            out_cp[b].wait()


def _sc_body(p_hbm, a_hbm, f_hbm, out_hbm, buf0, buf1, buf2, buf3,
             si0, si1, si2, si3, so0, so1, so2, so3):
    wid = lax.axis_index("s") * 2 + lax.axis_index("c")
    bufs = (buf0, buf1, buf2, buf3)
    sin = (si0, si1, si2, si3)
    son = (so0, so1, so2, so3)
    pb_w = _N_PAPER // _ROWS_PER_W
    ab_w = _N_AUTHOR // _ROWS_PER_W

    @pl.when(wid < pb_w)
    def _():
        _copy_range(p_hbm, out_hbm, wid * _ROWS_PER_W, wid * _ROWS_PER_W,
                    bufs, sin, son)

    @pl.when(jnp.logical_and(wid >= pb_w, wid < pb_w + ab_w))
    def _():
        r = (wid - pb_w) * _ROWS_PER_W
        _copy_range(a_hbm, out_hbm, r, _N_PAPER + r, bufs, sin, son)

    @pl.when(wid >= pb_w + ab_w)
    def _():
        r = (wid - pb_w - ab_w) * _ROWS_PER_W
        _copy_range(f_hbm, out_hbm, r, _N_PAPER + _N_AUTHOR + r,
                    bufs, sin, son)


def kernel(embed_paper, embed_author, embed_field):
    k = pl.kernel(
        _sc_body,
        out_type=jax.ShapeDtypeStruct((_TOTAL, _EMBED), jnp.float32),
        mesh=plsc.VectorSubcoreMesh(core_axis_name="c", subcore_axis_name="s"),
        scratch_types=(
            [pltpu.VMEM((_CH, _EMBED), jnp.float32)] * _NBUF
            + [pltpu.SemaphoreType.DMA] * (2 * _NBUF)
        ),
    )
    return k(embed_paper, embed_author, embed_field)


# speedup vs baseline: 1.0726x; 1.0074x over previous
import jax
import jax.numpy as jnp
from jax import lax
from jax.experimental import pallas as pl
from jax.experimental.pallas import tpu as pltpu
from jax.experimental.pallas import tpu_sc as plsc

_N_PAPER = 100000
_N_AUTHOR = 50000
_N_FIELD = 10000
_EMBED = 128
_TOTAL = _N_PAPER + _N_AUTHOR + _N_FIELD
_NW = 32
_ROWS_PER_W = _TOTAL // _NW
_CH = 200
_NCH = _ROWS_PER_W // _CH
_NBUF = 5
_LOOKAHEAD = 3


def _copy_range(src, out, sbase, obase, bufs, sin, son):
    in_cp = [None] * _NBUF
    out_cp = [None] * _NBUF

    def make_in(c):
        b = c % _NBUF
        cp = pltpu.make_async_copy(
            src.at[pl.ds(sbase + c * _CH, _CH)], bufs[b], sin[b])
        cp.start()
        in_cp[b] = cp

    for c in range(min(_LOOKAHEAD, _NCH)):
        make_in(c)
    for c in range(_NCH):
        b = c % _NBUF
        in_cp[b].wait()
        ocp = pltpu.make_async_copy(
            bufs[b], out.at[pl.ds(obase + c * _CH, _CH)], son[b])
        ocp.start()
        out_cp[b] = ocp
        n = c + _LOOKAHEAD
        if n < _NCH:
            nb = n % _NBUF
            if out_cp[nb] is not None:
                out_cp[nb].wait()
                out_cp[nb] = None
            make_in(n)
    for b in range(_NBUF):
        if out_cp[b] is not None:
            out_cp[b].wait()


def _sc_body(p_hbm, a_hbm, f_hbm, out_hbm, buf0, buf1, buf2, buf3, buf4,
             si0, si1, si2, si3, si4, so0, so1, so2, so3, so4):
    wid = lax.axis_index("s") * 2 + lax.axis_index("c")
    bufs = (buf0, buf1, buf2, buf3, buf4)
    sin = (si0, si1, si2, si3, si4)
    son = (so0, so1, so2, so3, so4)
    pb_w = _N_PAPER // _ROWS_PER_W
    ab_w = _N_AUTHOR // _ROWS_PER_W

    @pl.when(wid < pb_w)
    def _():
        _copy_range(p_hbm, out_hbm, wid * _ROWS_PER_W, wid * _ROWS_PER_W,
                    bufs, sin, son)

    @pl.when(jnp.logical_and(wid >= pb_w, wid < pb_w + ab_w))
    def _():
        r = (wid - pb_w) * _ROWS_PER_W
        _copy_range(a_hbm, out_hbm, r, _N_PAPER + r, bufs, sin, son)

    @pl.when(wid >= pb_w + ab_w)
    def _():
        r = (wid - pb_w - ab_w) * _ROWS_PER_W
        _copy_range(f_hbm, out_hbm, r, _N_PAPER + _N_AUTHOR + r,
                    bufs, sin, son)


def kernel(embed_paper, embed_author, embed_field):
    k = pl.kernel(
        _sc_body,
        out_type=jax.ShapeDtypeStruct((_TOTAL, _EMBED), jnp.float32),
        mesh=plsc.VectorSubcoreMesh(core_axis_name="c", subcore_axis_name="s"),
        scratch_types=(
            [pltpu.VMEM((_CH, _EMBED), jnp.float32)] * _NBUF
            + [pltpu.SemaphoreType.DMA] * (2 * _NBUF)
        ),
    )
    return k(embed_paper, embed_author, embed_field)


# speedup vs baseline: 1.5981x; 1.4900x over previous
import jax
import jax.numpy as jnp
from jax.experimental import pallas as pl
from jax.experimental.pallas import tpu as pltpu

_N_PAPER = 100000
_N_AUTHOR = 50000
_N_FIELD = 10000
_EMBED = 128
_TOTAL = _N_PAPER + _N_AUTHOR + _N_FIELD
_CH = 10000
_NCH = _TOTAL // _CH
_NBUF = 6
_LOOKAHEAD = 4


def _src_for_chunk(c, p_ref, a_ref, f_ref):
    row = c * _CH
    if row < _N_PAPER:
        return p_ref, row
    if row < _N_PAPER + _N_AUTHOR:
        return a_ref, row - _N_PAPER
    return f_ref, row - _N_PAPER - _N_AUTHOR


def _concat_kernel(p_ref, a_ref, f_ref, o_ref, bufs, sin, son):
    in_cp = [None] * _NBUF
    out_cp = [None] * _NBUF

    def make_in(c):
        b = c % _NBUF
        src, off = _src_for_chunk(c, p_ref, a_ref, f_ref)
        cp = pltpu.make_async_copy(
            src.at[pl.ds(off, _CH)], bufs.at[b], sin.at[b])
        cp.start()
        in_cp[b] = cp

    for c in range(min(_LOOKAHEAD, _NCH)):
        make_in(c)
    for c in range(_NCH):
        b = c % _NBUF
        in_cp[b].wait()
        ocp = pltpu.make_async_copy(
            bufs.at[b], o_ref.at[pl.ds(c * _CH, _CH)], son.at[b])
        ocp.start()
        out_cp[b] = ocp
        n = c + _LOOKAHEAD
        if n < _NCH:
            nb = n % _NBUF
            if out_cp[nb] is not None:
                out_cp[nb].wait()
                out_cp[nb] = None
            make_in(n)
    for b in range(_NBUF):
        if out_cp[b] is not None:
            out_cp[b].wait()


def kernel(embed_paper, embed_author, embed_field):
    return pl.pallas_call(
        _concat_kernel,
        out_shape=jax.ShapeDtypeStruct((_TOTAL, _EMBED), jnp.float32),
        in_specs=[
            pl.BlockSpec(memory_space=pl.ANY),
            pl.BlockSpec(memory_space=pl.ANY),
            pl.BlockSpec(memory_space=pl.ANY),
        ],
        out_specs=pl.BlockSpec(memory_space=pl.ANY),
        scratch_shapes=[
            pltpu.VMEM((_NBUF, _CH, _EMBED), jnp.float32),
            pltpu.SemaphoreType.DMA((_NBUF,)),
            pltpu.SemaphoreType.DMA((_NBUF,)),
        ],
    )(embed_paper, embed_author, embed_field)
